# baseline (speedup 1.0000x reference)
import math

import jax
import jax.numpy as jnp
import numpy as np
from jax import lax
from jax.experimental import pallas as pl
from jax.experimental.pallas import tpu as pltpu
from jax.experimental.pallas import tpu_sc as plsc

_NUM_CORES = 2
_NUM_SUBCORES = 16
_NW = _NUM_CORES * _NUM_SUBCORES

_D_ID = 32
_D_MODEL = 64
_MAX_DAYS = 365
_N_FREQ = 8


def _sc_gather(table, idx, sigma, chunk):
    n = sigma.shape[0]
    d = table.shape[1]
    per_w = n // _NW
    n_chunks = per_w // chunk
    mesh = plsc.VectorSubcoreMesh(
        core_axis_name="c", subcore_axis_name="s",
        num_cores=_NUM_CORES, num_subcores=_NUM_SUBCORES,
    )

    def body(table_hbm, idx_hbm, sigma_hbm, out_hbm, sig_v, idx_v, rows_v, sem):
        wid = lax.axis_index("s") * _NUM_CORES + lax.axis_index("c")
        base = wid * per_w

        @pl.loop(0, n_chunks)
        def _chunk(c):
            off = base + c * chunk
            pltpu.sync_copy(sigma_hbm.at[pl.ds(off, chunk)], sig_v)
            pltpu.async_copy(idx_hbm.at[sig_v], idx_v, sem).wait()
            pltpu.async_copy(table_hbm.at[idx_v], rows_v, sem).wait()
            pltpu.sync_copy(rows_v, out_hbm.at[pl.ds(off, chunk)])

    f = pl.kernel(
        body,
        out_type=jax.ShapeDtypeStruct((n, d), jnp.float32),
        mesh=mesh,
        compiler_params=pltpu.CompilerParams(use_tc_tiling_on_sc=False),
        scratch_types=[
            pltpu.VMEM((chunk,), jnp.int32),
            pltpu.VMEM((chunk,), jnp.int32),
            pltpu.VMEM((chunk, d), jnp.float32),
            pltpu.SemaphoreType.DMA,
        ],
    )
    return f(table, idx, sigma)


def _tc_body(typ_ref, cat_ref, ctry_ref, day_ref, imp_ref, nv_ref, sp_ref,
             ma_ref, eid_ref, w4_ref, wall_ref, div_ref, lnt_ref, o_ref):
    nb = typ_ref.shape[2]
    qb = nb // 4
    typ = typ_ref[0]
    cat = cat_ref[0]
    ctry = ctry_ref[0]
    day = day_ref[0]

    it = lax.broadcasted_iota(jnp.int32, (20, nb), 0)
    oh = ((it == typ) | (it == cat + 6) | (it == ctry + 14)).astype(jnp.float32)

    dayf = jnp.clip(jnp.abs(day), 0, _MAX_DAYS - 1).astype(jnp.float32)
    ang = div_ref[...] * dayf
    nrot = (ang * jnp.float32(0.15915494309644432)
            + jnp.float32(0.5)).astype(jnp.int32).astype(jnp.float32)
    r = (ang - nrot * jnp.float32(6.28125)) - nrot * jnp.float32(0.0019353071795864769)
    r2 = r * r
    s = r * (jnp.float32(9.99984587e-01) + r2 * (jnp.float32(-1.66632582e-01)
        + r2 * (jnp.float32(8.31238293e-03) + r2 * (jnp.float32(-1.93161822e-04)
        + r2 * jnp.float32(2.17321007e-06)))))
    c = (jnp.float32(9.99999443e-01) + r2 * (jnp.float32(-4.99995580e-01)
        + r2 * (jnp.float32(4.16610316e-02) + r2 * (jnp.float32(-1.38627433e-03)
        + r2 * (jnp.float32(2.42531378e-05) + r2 * jnp.float32(-2.21936942e-07))))))

    fv = jnp.concatenate([imp_ref[0], nv_ref[0], sp_ref[0], ma_ref[0]], axis=0)
    ones = jnp.ones((1, nb), jnp.float32)
    feats = jnp.concatenate([oh, s, c, fv, ones], axis=0)

    fpart = lax.dot_general(
        wall_ref[...], feats, (((0,), (0,)), ((), ())),
        preferred_element_type=jnp.float32,
    )

    eid_t = jnp.transpose(eid_ref[...], (1, 0))
    c4 = lax.dot_general(
        w4_ref[...], eid_t, (((0,), (0,)), ((), ())),
        preferred_element_type=jnp.float32,
    )

    g = lnt_ref[:, 0:1]
    b = lnt_ref[:, 1:2]
    one_row = jnp.ones((1, _D_MODEL), jnp.float32)
    for j in range(4):
        dlt = fpart[:, j * qb:(j + 1) * qb] + c4[j * 64:(j + 1) * 64, :]
        sq = dlt * dlt
        v = lax.dot_general(
            one_row, sq, (((1,), (0,)), ((), ())),
            preferred_element_type=jnp.float32,
        ) * (1.0 / _D_MODEL)
        xh = dlt * lax.rsqrt(v + 1e-5)
        o_ref[0, :, j * qb:(j + 1) * qb] = xh * g + b


def kernel(indicator_ids, pub_type_ids, category_ids, country_ids, importance,
           days_offset, normalized_value, surprise, ma5,
           identity_emb, type_emb, category_emb, country_emb,
           imp_W, imp_b, proj_W, proj_b, ln_g, ln_b):
    bsz, ssz = indicator_ids.shape
    n = bsz * ssz
    qb = bsz // 4

    u = np.arange(n)
    rr = u % bsz
    sigma = jnp.asarray(
        (u - rr + (rr % 4) * qb + rr // 4).astype(np.int32))
    idx_flat = indicator_ids.T.reshape(-1)
    nsplit = 8
    npart = n // nsplit
    prows = npart * _D_ID // 128
    eid_parts = [
        _sc_gather(identity_emb, idx_flat,
                   sigma[k * npart:(k + 1) * npart], chunk=3200)
        .reshape(-1).reshape(prows, 128)
        for k in range(nsplit)
    ]

    w_id = proj_W[:, 0:32].T
    t_type = type_emb @ proj_W[:, 32:40].T
    t_cat = category_emb @ proj_W[:, 40:56].T
    t_ctry = country_emb @ proj_W[:, 56:64].T
    w_imp = proj_W[:, 64:72]
    v_imp = w_imp @ imp_W[:, 0]
    const = proj_b + w_imp @ imp_b
    w_temp = proj_W[:, 72:88]
    w_sin = w_temp[:, 0::2].T
    w_cos = w_temp[:, 1::2].T
    w_num = proj_W[:, 88:91].T
    w_all = jnp.concatenate(
        [t_type, t_cat, t_ctry, w_sin, w_cos,
         v_imp[None, :], w_num, const[None, :]], axis=0)
    w_id = w_id - jnp.mean(w_id, axis=1, keepdims=True)
    w_all = w_all - jnp.mean(w_all, axis=1, keepdims=True)
    w4 = jnp.kron(jnp.eye(4, dtype=jnp.float32), w_id)

    div = np.exp(np.arange(0, 2 * _N_FREQ, 2, dtype=np.float32)
                 * (-math.log(10000.0) / (2 * _N_FREQ))).astype(np.float32)
    div = jnp.asarray(div).reshape(_N_FREQ, 1)

    lnt = jnp.stack([ln_g, ln_b], axis=1)

    def tview(x):
        return x.T.reshape(ssz, 1, bsz)

    full = lambda shape: pl.BlockSpec(shape, lambda i: tuple(0 for _ in shape))
    scalars = (tview(pub_type_ids), tview(category_ids), tview(country_ids),
               tview(days_offset), tview(importance), tview(normalized_value),
               tview(surprise), tview(ma5))
    weights = (w4, w_all, div, lnt)
    wspecs = [full((128, 256)), full((41, _D_MODEL)),
              full((_N_FREQ, 1)), full((_D_MODEL, 2))]
    out_shape = jax.ShapeDtypeStruct((ssz, _D_MODEL, bsz), jnp.float32)

    def tc_part(eid_part, offs, prev):
        svec = pl.BlockSpec((1, 1, bsz), lambda i: (i + offs, 0, 0))
        especs = [svec] * 8 + [
            pl.BlockSpec((bsz * _D_ID // 128, 128), lambda i: (i, 0))
        ] + wspecs
        args = scalars + (eid_part,) + weights
        if prev is None:
            return pl.pallas_call(
                _tc_body,
                grid=(ssz // nsplit,),
                in_specs=especs,
                out_specs=pl.BlockSpec((1, _D_MODEL, bsz),
                                       lambda i: (i + offs, 0, 0)),
                out_shape=out_shape,
            )(*args)

        def body2(*refs):
            _tc_body(*refs[:13], refs[14])

        return pl.pallas_call(
            body2,
            grid=(ssz // nsplit,),
            in_specs=especs + [pl.BlockSpec(memory_space=pl.ANY)],
            out_specs=pl.BlockSpec((1, _D_MODEL, bsz),
                                   lambda i: (i + offs, 0, 0)),
            out_shape=out_shape,
            input_output_aliases={13: 0},
        )(*args, prev)

    out = None
    for k in range(nsplit):
        out = tc_part(eid_parts[k], k * (ssz // nsplit), out)

    return out.transpose(2, 0, 1)


# speedup vs baseline: 1.0245x; 1.0245x over previous
import math

import jax
import jax.numpy as jnp
import numpy as np
from jax import lax
from jax.experimental import pallas as pl
from jax.experimental.pallas import tpu as pltpu
from jax.experimental.pallas import tpu_sc as plsc

_NUM_CORES = 2
_NUM_SUBCORES = 16
_NW = _NUM_CORES * _NUM_SUBCORES

_D_ID = 32
_D_MODEL = 64
_MAX_DAYS = 365
_N_FREQ = 8


def _sc_gather(table, idx, sigma, chunk):
    n = sigma.shape[0]
    d = table.shape[1]
    per_w = n // _NW
    n_chunks = per_w // chunk
    mesh = plsc.VectorSubcoreMesh(
        core_axis_name="c", subcore_axis_name="s",
        num_cores=_NUM_CORES, num_subcores=_NUM_SUBCORES,
    )

    def body(table_hbm, idx_hbm, sigma_hbm, out_hbm, sig_v, idx_v, rows_v, sem):
        wid = lax.axis_index("s") * _NUM_CORES + lax.axis_index("c")
        base = wid * per_w

        @pl.loop(0, n_chunks)
        def _chunk(c):
            off = base + c * chunk
            pltpu.sync_copy(sigma_hbm.at[pl.ds(off, chunk)], sig_v)
            pltpu.async_copy(idx_hbm.at[sig_v], idx_v, sem).wait()
            pltpu.async_copy(table_hbm.at[idx_v], rows_v, sem).wait()
            pltpu.sync_copy(rows_v, out_hbm.at[pl.ds(off, chunk)])

    f = pl.kernel(
        body,
        out_type=jax.ShapeDtypeStruct((n, d), jnp.float32),
        mesh=mesh,
        compiler_params=pltpu.CompilerParams(use_tc_tiling_on_sc=False),
        scratch_types=[
            pltpu.VMEM((chunk,), jnp.int32),
            pltpu.VMEM((chunk,), jnp.int32),
            pltpu.VMEM((chunk, d), jnp.float32),
            pltpu.SemaphoreType.DMA,
        ],
    )
    return f(table, idx, sigma)


def _tc_body(typ_ref, cat_ref, ctry_ref, day_ref, imp_ref, nv_ref, sp_ref,
             ma_ref, eid_ref, w4_ref, wall_ref, div_ref, lnt_ref, o_ref):
    nb = typ_ref.shape[2]
    qb = nb // 4
    typ = typ_ref[0]
    cat = cat_ref[0]
    ctry = ctry_ref[0]
    day = day_ref[0]

    it = lax.broadcasted_iota(jnp.int32, (20, nb), 0)
    oh = ((it == typ) | (it == cat + 6) | (it == ctry + 14)).astype(jnp.float32)

    dayf = jnp.clip(jnp.abs(day), 0, _MAX_DAYS - 1).astype(jnp.float32)
    ang = div_ref[...] * dayf
    nrot = (ang * jnp.float32(0.15915494309644432)
            + jnp.float32(0.5)).astype(jnp.int32).astype(jnp.float32)
    r = (ang - nrot * jnp.float32(6.28125)) - nrot * jnp.float32(0.0019353071795864769)
    r2 = r * r
    s = r * (jnp.float32(9.99984587e-01) + r2 * (jnp.float32(-1.66632582e-01)
        + r2 * (jnp.float32(8.31238293e-03) + r2 * (jnp.float32(-1.93161822e-04)
        + r2 * jnp.float32(2.17321007e-06)))))
    c = (jnp.float32(9.99999443e-01) + r2 * (jnp.float32(-4.99995580e-01)
        + r2 * (jnp.float32(4.16610316e-02) + r2 * (jnp.float32(-1.38627433e-03)
        + r2 * (jnp.float32(2.42531378e-05) + r2 * jnp.float32(-2.21936942e-07))))))

    fv = jnp.concatenate([imp_ref[0], nv_ref[0], sp_ref[0], ma_ref[0]], axis=0)
    ones = jnp.ones((1, nb), jnp.float32)
    feats = jnp.concatenate([oh, s, c, fv, ones], axis=0)

    fpart = lax.dot_general(
        wall_ref[...], feats, (((0,), (0,)), ((), ())),
        preferred_element_type=jnp.float32,
    )

    eid_t = jnp.transpose(eid_ref[...], (1, 0))
    c4 = lax.dot_general(
        w4_ref[...], eid_t, (((0,), (0,)), ((), ())),
        preferred_element_type=jnp.float32,
    )

    g = lnt_ref[:, 0:1]
    b = lnt_ref[:, 1:2]
    one_row = jnp.ones((1, _D_MODEL), jnp.float32)
    for j in range(4):
        dlt = fpart[:, j * qb:(j + 1) * qb] + c4[j * 64:(j + 1) * 64, :]
        sq = dlt * dlt
        v = lax.dot_general(
            one_row, sq, (((1,), (0,)), ((), ())),
            preferred_element_type=jnp.float32,
        ) * (1.0 / _D_MODEL)
        xh = dlt * lax.rsqrt(v + 1e-5)
        o_ref[0, :, j * qb:(j + 1) * qb] = xh * g + b


def kernel(indicator_ids, pub_type_ids, category_ids, country_ids, importance,
           days_offset, normalized_value, surprise, ma5,
           identity_emb, type_emb, category_emb, country_emb,
           imp_W, imp_b, proj_W, proj_b, ln_g, ln_b):
    bsz, ssz = indicator_ids.shape
    n = bsz * ssz
    qb = bsz // 4

    u = np.arange(n)
    rr = u % bsz
    sigma = jnp.asarray(
        (u - rr + (rr % 4) * qb + rr // 4).astype(np.int32))
    idx_flat = indicator_ids.T.reshape(-1)
    nsplit = 4
    npart = n // nsplit
    prows = npart * _D_ID // 128
    eid_parts = [
        _sc_gather(identity_emb, idx_flat,
                   sigma[k * npart:(k + 1) * npart], chunk=3200)
        .reshape(-1).reshape(prows, 128)
        for k in range(nsplit)
    ]

    w_id = proj_W[:, 0:32].T
    t_type = type_emb @ proj_W[:, 32:40].T
    t_cat = category_emb @ proj_W[:, 40:56].T
    t_ctry = country_emb @ proj_W[:, 56:64].T
    w_imp = proj_W[:, 64:72]
    v_imp = w_imp @ imp_W[:, 0]
    const = proj_b + w_imp @ imp_b
    w_temp = proj_W[:, 72:88]
    w_sin = w_temp[:, 0::2].T
    w_cos = w_temp[:, 1::2].T
    w_num = proj_W[:, 88:91].T
    w_all = jnp.concatenate(
        [t_type, t_cat, t_ctry, w_sin, w_cos,
         v_imp[None, :], w_num, const[None, :]], axis=0)
    w_id = w_id - jnp.mean(w_id, axis=1, keepdims=True)
    w_all = w_all - jnp.mean(w_all, axis=1, keepdims=True)
    w4 = jnp.kron(jnp.eye(4, dtype=jnp.float32), w_id)

    div = np.exp(np.arange(0, 2 * _N_FREQ, 2, dtype=np.float32)
                 * (-math.log(10000.0) / (2 * _N_FREQ))).astype(np.float32)
    div = jnp.asarray(div).reshape(_N_FREQ, 1)

    lnt = jnp.stack([ln_g, ln_b], axis=1)

    def tview(x):
        return x.T.reshape(ssz, 1, bsz)

    full = lambda shape: pl.BlockSpec(shape, lambda i: tuple(0 for _ in shape))
    scalars = (tview(pub_type_ids), tview(category_ids), tview(country_ids),
               tview(days_offset), tview(importance), tview(normalized_value),
               tview(surprise), tview(ma5))
    weights = (w4, w_all, div, lnt)
    wspecs = [full((128, 256)), full((41, _D_MODEL)),
              full((_N_FREQ, 1)), full((_D_MODEL, 2))]
    out_shape = jax.ShapeDtypeStruct((ssz, _D_MODEL, bsz), jnp.float32)

    def tc_part(eid_part, offs, prev):
        svec = pl.BlockSpec((1, 1, bsz), lambda i: (i + offs, 0, 0))
        especs = [svec] * 8 + [
            pl.BlockSpec((bsz * _D_ID // 128, 128), lambda i: (i, 0))
        ] + wspecs
        args = scalars + (eid_part,) + weights
        if prev is None:
            return pl.pallas_call(
                _tc_body,
                grid=(ssz // nsplit,),
                in_specs=especs,
                out_specs=pl.BlockSpec((1, _D_MODEL, bsz),
                                       lambda i: (i + offs, 0, 0)),
                out_shape=out_shape,
            )(*args)

        def body2(*refs):
            _tc_body(*refs[:13], refs[14])

        return pl.pallas_call(
            body2,
            grid=(ssz // nsplit,),
            in_specs=especs + [pl.BlockSpec(memory_space=pl.ANY)],
            out_specs=pl.BlockSpec((1, _D_MODEL, bsz),
                                   lambda i: (i + offs, 0, 0)),
            out_shape=out_shape,
            input_output_aliases={13: 0},
        )(*args, prev)

    out = None
    for k in range(nsplit):
        out = tc_part(eid_parts[k], k * (ssz // nsplit), out)

    return out.transpose(2, 0, 1)
